# Initial kernel scaffold; baseline (speedup 1.0000x reference)
import jax
import jax.numpy as jnp
from jax.experimental import pallas as pl


def kernel(x0, x1, edge_index0, edge_index1, ptr0, ptr1, params):
    raise NotImplementedError("write your pallas kernel here")


# speedup vs baseline: 169.3832x; 169.3832x over previous
import functools

import jax
import jax.numpy as jnp
from jax import lax
from jax.experimental import pallas as pl
from jax.experimental.pallas import tpu as pltpu
from jax.experimental.pallas import tpu_sc as plsc

N = 10000
NPAD = 10240
E = 320000
NG = 2
B = 64
HL = 8
NTILES = 16
EP_T = E // NTILES
ECH = 10000
NCH = EP_T // ECH
NP_T = NPAD // NTILES
ROWS = 1024
NBLK = NPAD // ROWS


def _proj_body(x0_ref, x1_ref, w0_ref, w1_ref, p0_ref, p1_ref,
               hsq_ref, cmax_ref):
    i = pl.program_id(0)
    a0 = lax.dot_general(w0_ref[...], p0_ref[...], (((1,), (0,)), ((), ())),
                         preferred_element_type=jnp.float32,
                         precision=lax.Precision.HIGHEST)
    a1 = lax.dot_general(w1_ref[...], p1_ref[...], (((1,), (0,)), ((), ())),
                         preferred_element_type=jnp.float32,
                         precision=lax.Precision.HIGHEST)
    r0 = lax.dot_general(a0, x0_ref[...], (((0,), (1,)), ((), ())),
                         preferred_element_type=jnp.float32,
                         precision=lax.Precision.HIGHEST)
    r1 = lax.dot_general(a1, x1_ref[...], (((0,), (1,)), ((), ())),
                         preferred_element_type=jnp.float32,
                         precision=lax.Precision.HIGHEST)
    valid = (lax.broadcasted_iota(jnp.int32, (3, ROWS), 1) + i * ROWS) < N
    r0z = jnp.where(valid, r0, 0.0)
    r1z = jnp.where(valid, r1, 0.0)
    hsq_ref[0] = r0z
    hsq_ref[1] = r1z
    neg = jnp.float32(-1e30)
    m = jnp.concatenate(
        [jnp.max(jnp.where(valid, r0, neg), axis=1, keepdims=True),
         jnp.max(jnp.where(valid, r1, neg), axis=1, keepdims=True)],
        axis=1)

    @pl.when(i == 0)
    def _():
        cmax_ref[...] = m

    @pl.when(i > 0)
    def _():
        cmax_ref[...] = jnp.maximum(cmax_ref[...], m)


_proj = pl.pallas_call(
    _proj_body,
    grid=(NBLK,),
    in_specs=[
        pl.BlockSpec((ROWS, 128), lambda i: (i, 0)),
        pl.BlockSpec((ROWS, 128), lambda i: (i, 0)),
        pl.BlockSpec((128, 16), lambda i: (0, 0)),
        pl.BlockSpec((128, 16), lambda i: (0, 0)),
        pl.BlockSpec((16, 3), lambda i: (0, 0)),
        pl.BlockSpec((16, 3), lambda i: (0, 0)),
    ],
    out_specs=[
        pl.BlockSpec((NG, 3, ROWS), lambda i: (0, 0, i)),
        pl.BlockSpec((3, NG), lambda i: (0, 0)),
    ],
    out_shape=[
        jax.ShapeDtypeStruct((NG, 3, NPAD), jnp.float32),
        jax.ShapeDtypeStruct((3, NG), jnp.float32),
    ],
)


def _sc_body(hsq, consts, s0, d0, s1, d1, go_out,
             hs_t, hd_t, q_t, src_t, dst_t, num_t, den_t,
             accn, accd, rb, go_t, cst_t, nums_sh, dens_sh):
    c = lax.axis_index("c")
    s = lax.axis_index("s")

    hb = c * (3 * NPAD)
    pltpu.sync_copy(hsq.at[pl.ds(hb, NPAD)], hs_t)
    pltpu.sync_copy(hsq.at[pl.ds(hb + NPAD, NPAD)], hd_t)
    pltpu.sync_copy(hsq.at[pl.ds(hb + 2 * NPAD, NPAD)], q_t)
    pltpu.sync_copy(consts.at[pl.ds(c * 32, 32)], cst_t)

    cvec = cst_t[pl.ds(0, 16)]
    bvec = cst_t[pl.ds(16, 16)]

    zero16 = jnp.zeros((16,), jnp.float32)

    def zero_body(i, carry):
        sl = pl.ds(i * 16, 16)
        num_t[sl] = zero16
        den_t[sl] = zero16
        return carry

    lax.fori_loop(0, NPAD // 16, zero_body, 0)

    eb = s * EP_T

    def edge_body(i, carry):
        sl = pl.ds(i * 16, 16)
        sidx = src_t[sl]
        didx = dst_t[sl]
        av = plsc.load_gather(hs_t, [sidx])
        bv = plsc.load_gather(hd_t, [didx])
        e = av + bv
        e = jnp.maximum(e, e * 0.2)
        ex = jnp.exp(e - cvec)
        qs = plsc.load_gather(q_t, [sidx])
        plsc.addupdate_scatter(den_t, [didx], ex)
        plsc.addupdate_scatter(num_t, [didx], ex * qs)
        return carry

    for k in range(NCH):
        cb = eb + k * ECH

        @pl.when(c == 0)
        def _():
            pltpu.sync_copy(s0.at[pl.ds(cb, ECH)], src_t)
            pltpu.sync_copy(d0.at[pl.ds(cb, ECH)], dst_t)

        @pl.when(c == 1)
        def _():
            pltpu.sync_copy(s1.at[pl.ds(cb, ECH)], src_t)
            pltpu.sync_copy(d1.at[pl.ds(cb, ECH)], dst_t)

        lax.fori_loop(0, ECH // 16, edge_body, 0)

    pltpu.sync_copy(num_t, nums_sh.at[s])
    pltpu.sync_copy(den_t, dens_sh.at[s])
    plsc.subcore_barrier()

    nb = s * NP_T

    def zacc_body(j, carry):
        sl = pl.ds(j * 16, 16)
        accn[sl] = zero16
        accd[sl] = zero16
        return carry

    lax.fori_loop(0, NP_T // 16, zacc_body, 0)

    for r in range(NTILES):
        pltpu.sync_copy(nums_sh.at[r, pl.ds(nb, NP_T)], rb)

        def addn_body(j, carry):
            sl = pl.ds(j * 16, 16)
            accn[sl] = accn[sl] + rb[sl]
            return carry

        lax.fori_loop(0, NP_T // 16, addn_body, 0)
        pltpu.sync_copy(dens_sh.at[r, pl.ds(nb, NP_T)], rb)

        def addd_body(j, carry):
            sl = pl.ds(j * 16, 16)
            accd[sl] = accd[sl] + rb[sl]
            return carry

        lax.fori_loop(0, NP_T // 16, addd_body, 0)

    def node_body(j, carry):
        sl = pl.ds(j * 16, 16)
        gsl = pl.ds(nb + j * 16, 16)
        hv = hs_t[gsl]
        dv = hd_t[gsl]
        qv = q_t[gsl]
        es = hv + dv
        es = jnp.maximum(es, es * 0.2)
        exs = jnp.exp(es - cvec)
        dtot = accd[sl] + exs
        ntot = accn[sl] + exs * qv
        dtot = jnp.where(dtot == 0.0, 1.0, dtot)
        go_t[sl] = ntot / dtot + bvec
        return carry

    lax.fori_loop(0, NP_T // 16, node_body, 0)
    pltpu.sync_copy(go_t, go_out.at[pl.ds(c * NPAD + nb, NP_T)])


_sc = pl.kernel(
    _sc_body,
    out_type=jax.ShapeDtypeStruct((NG * NPAD,), jnp.float32),
    mesh=plsc.VectorSubcoreMesh(core_axis_name="c", subcore_axis_name="s"),
    compiler_params=pltpu.CompilerParams(needs_layout_passes=False),
    scratch_types=[
        pltpu.VMEM((NPAD,), jnp.float32),
        pltpu.VMEM((NPAD,), jnp.float32),
        pltpu.VMEM((NPAD,), jnp.float32),
        pltpu.VMEM((ECH,), jnp.int32),
        pltpu.VMEM((ECH,), jnp.int32),
        pltpu.VMEM((NPAD,), jnp.float32),
        pltpu.VMEM((NPAD,), jnp.float32),
        pltpu.VMEM((NP_T,), jnp.float32),
        pltpu.VMEM((NP_T,), jnp.float32),
        pltpu.VMEM((NP_T,), jnp.float32),
        pltpu.VMEM((NP_T,), jnp.float32),
        pltpu.VMEM((32,), jnp.float32),
        pltpu.VMEM_SHARED((NTILES, NPAD), jnp.float32),
        pltpu.VMEM_SHARED((NTILES, NPAD), jnp.float32),
    ],
)


def _post_body(go_ref, plo_ref, phi_ref, winv_ref,
               w0_ref, b0_ref, wi_ref, bi_ref, wf_ref, bf_ref, out_ref):
    io = lax.broadcasted_iota(jnp.int32, (B, NPAD), 1)
    ps = []
    for g in range(NG):
        lo = plo_ref[g]
        hi = phi_ref[g]
        wv = winv_ref[g]
        m = jnp.where((io >= lo) & (io < hi), wv, 0.0)
        gv = go_ref[g:g + 1]
        ps.append(jnp.sum(m * gv, axis=1, keepdims=True))
    pcat = jnp.concatenate(ps, axis=1)

    weff = w0_ref[...]
    beff = b0_ref[...]
    for j in range(HL):
        wj = wi_ref[j]
        weff = lax.dot_general(weff, wj, (((1,), (0,)), ((), ())),
                               preferred_element_type=jnp.float32,
                         precision=lax.Precision.HIGHEST)
        beff = lax.dot_general(beff, wj, (((1,), (0,)), ((), ())),
                               preferred_element_type=jnp.float32,
                         precision=lax.Precision.HIGHEST) + bi_ref[j:j + 1]
    weff = lax.dot_general(weff, wf_ref[...], (((1,), (0,)), ((), ())),
                           preferred_element_type=jnp.float32,
                         precision=lax.Precision.HIGHEST)
    beff = lax.dot_general(beff, wf_ref[...], (((1,), (0,)), ((), ())),
                           preferred_element_type=jnp.float32,
                         precision=lax.Precision.HIGHEST) + bf_ref[...]
    out_ref[...] = lax.dot_general(pcat, weff, (((1,), (0,)), ((), ())),
                                   preferred_element_type=jnp.float32,
                         precision=lax.Precision.HIGHEST) + beff


_post = pl.pallas_call(
    _post_body,
    out_shape=jax.ShapeDtypeStruct((B, NG), jnp.float32),
)


def kernel(x0, x1, edge_index0, edge_index1, ptr0, ptr1, params):
    f32 = jnp.float32
    p0 = jnp.stack([params["gat_asrc"][0], params["gat_adst"][0],
                    params["post_w"][0][:, 0]], axis=1)
    p1 = jnp.stack([params["gat_asrc"][1], params["gat_adst"][1],
                    params["post_w"][1][:, 0]], axis=1)

    hsq, cmax = _proj(x0.astype(f32), x1.astype(f32),
                      params["gat_W"][0], params["gat_W"][1], p0, p1)

    cval = cmax[0] + cmax[1]
    cbias = (jnp.einsum("gh,gh->g", params["gat_b"],
                        params["post_w"][:, :, 0]) + params["post_b"][:, 0])
    consts = jnp.broadcast_to(
        jnp.stack([cval, cbias], axis=1)[:, :, None], (NG, 2, 16)
    ).astype(f32).reshape(-1)

    ei0 = edge_index0.astype(jnp.int32)
    ei1 = edge_index1.astype(jnp.int32)
    go = _sc(hsq.reshape(-1), consts,
             ei0[0].reshape(-1), ei0[1].reshape(-1),
             ei1[0].reshape(-1), ei1[1].reshape(-1))
    go = go.reshape(NG, NPAD)

    ptr_s = jnp.stack([ptr0, ptr1]).astype(jnp.int32)
    plo = ptr_s[:, :B][:, :, None]
    phi = ptr_s[:, 1:][:, :, None]
    winv = 1.0 / (phi - plo).astype(f32)

    return _post(go, plo, phi, winv,
                 params["mlp_w0"], params["mlp_b0"].reshape(1, 16),
                 params["mlp_wi"], params["mlp_bi"],
                 params["mlp_wf"], params["mlp_bf"].reshape(1, NG))
